# Initial kernel scaffold; baseline (speedup 1.0000x reference)
import jax
import jax.numpy as jnp
from jax.experimental import pallas as pl


def kernel(n1_indices, n2_indices, W):
    raise NotImplementedError("write your pallas kernel here")


# speedup vs baseline: 2.4067x; 2.4067x over previous
import functools

import jax
import jax.numpy as jnp
from jax import lax
from jax.experimental import pallas as pl
from jax.experimental.pallas import tpu as pltpu
from jax.experimental.pallas import tpu_sc as plsc

E = 500000
D = 128
B = 128
NUM_BLOCKS = (E + B - 1) // B
LAST_BASE = E - B
NC = 2
NS = 16
NW = NC * NS
ITERS = (NUM_BLOCKS + NW - 1) // NW


def _sc_body(n1_hbm, n2_hbm, w_hbm, out_hbm,
             idx1_v, idx2_v, rows1_v, rows2_v, out_v, sem1, sem2):
  wid = lax.axis_index("s") * NC + lax.axis_index("c")

  def block_body(t, carry):
    block = wid + t * NW

    @pl.when(block < NUM_BLOCKS)
    def _():
      base = jnp.minimum(block * B, LAST_BASE)
      base = pl.multiple_of(base, 8)
      pltpu.sync_copy(n1_hbm.at[pl.ds(base, B)], idx1_v)
      pltpu.sync_copy(n2_hbm.at[pl.ds(base, B)], idx2_v)
      cp1 = pltpu.async_copy(w_hbm.at[idx1_v], rows1_v, sem1)
      cp2 = pltpu.async_copy(w_hbm.at[idx2_v], rows2_v, sem2)
      cp1.wait()
      cp2.wait()

      lane = lax.iota(jnp.int32, 16)

      def group_body(g, c):
        base_j = g * 16
        vec = jnp.zeros((16,), jnp.float32)
        for l in range(16):
          j = base_j + l
          acc = rows1_v[j, pl.ds(0, 16)] * rows2_v[j, pl.ds(0, 16)]
          for k in range(1, D // 16):
            acc = acc + (rows1_v[j, pl.ds(k * 16, 16)]
                         * rows2_v[j, pl.ds(k * 16, 16)])
          s = jnp.sum(acc)
          vec = jnp.where(lane == l, s, vec)
        out_v[pl.ds(base_j, 16)] = vec
        return c

      lax.fori_loop(0, B // 16, group_body, 0)
      pltpu.sync_copy(out_v, out_hbm.at[pl.ds(base, B)])

    return carry

  lax.fori_loop(0, ITERS, block_body, 0)


@jax.jit
def _run(n1, n2, w):
  mesh = plsc.VectorSubcoreMesh(core_axis_name="c", subcore_axis_name="s")
  f = pl.kernel(
      _sc_body,
      mesh=mesh,
      compiler_params=pltpu.CompilerParams(needs_layout_passes=False),
      out_type=jax.ShapeDtypeStruct((E,), jnp.float32),
      scratch_types=[
          pltpu.VMEM((B,), jnp.int32),
          pltpu.VMEM((B,), jnp.int32),
          pltpu.VMEM((B, D), jnp.float32),
          pltpu.VMEM((B, D), jnp.float32),
          pltpu.VMEM((B,), jnp.float32),
          pltpu.SemaphoreType.DMA,
          pltpu.SemaphoreType.DMA,
      ],
  )
  return f(n1, n2, w)


def kernel(n1_indices, n2_indices, W):
  n1 = n1_indices.astype(jnp.int32)
  n2 = n2_indices.astype(jnp.int32)
  return _run(n1, n2, W)


# speedup vs baseline: 3.1768x; 1.3200x over previous
import functools

import jax
import jax.numpy as jnp
from jax import lax
from jax.experimental import pallas as pl
from jax.experimental.pallas import tpu as pltpu
from jax.experimental.pallas import tpu_sc as plsc

E = 500000
D = 128
B = 128
NUM_BLOCKS = (E + B - 1) // B
LAST_BASE = E - B
NC = 2
NS = 16
NW = NC * NS
ITERS = (NUM_BLOCKS + NW - 1) // NW


def _sc_body(n1_hbm, n2_hbm, w_hbm, out_hbm,
             idx1a, idx1b, idx2a, idx2b,
             rows1a, rows1b, rows2a, rows2b, out_v,
             sem1a, sem1b, sem2a, sem2b):
  wid = lax.axis_index("s") * NC + lax.axis_index("c")
  idx1 = (idx1a, idx1b)
  idx2 = (idx2a, idx2b)
  rows1 = (rows1a, rows1b)
  rows2 = (rows2a, rows2b)
  sem1 = (sem1a, sem1b)
  sem2 = (sem2a, sem2b)

  def block_base(block):
    return pl.multiple_of(jnp.minimum(block * B, LAST_BASE), 8)

  def prefetch(block, b):
    base = block_base(block)
    pltpu.sync_copy(n1_hbm.at[pl.ds(base, B)], idx1[b])
    pltpu.sync_copy(n2_hbm.at[pl.ds(base, B)], idx2[b])
    pltpu.async_copy(w_hbm.at[idx1[b]], rows1[b], sem1[b])
    pltpu.async_copy(w_hbm.at[idx2[b]], rows2[b], sem2[b])

  prefetch(wid, 0)

  lane = lax.iota(jnp.int32, 16)

  def step(t, b):
    block = wid + t * NW
    nxt = block + NW

    @pl.when(nxt < NUM_BLOCKS)
    def _():
      prefetch(nxt, 1 - b)

    @pl.when(block < NUM_BLOCKS)
    def _():
      pltpu.make_async_copy(w_hbm.at[idx1[b]], rows1[b], sem1[b]).wait()
      pltpu.make_async_copy(w_hbm.at[idx2[b]], rows2[b], sem2[b]).wait()
      r1, r2 = rows1[b], rows2[b]

      def group_body(g, c):
        base_j = g * 16
        vec = jnp.zeros((16,), jnp.float32)
        for l in range(16):
          j = base_j + l
          acc = r1[j, pl.ds(0, 16)] * r2[j, pl.ds(0, 16)]
          for k in range(1, D // 16):
            acc = acc + r1[j, pl.ds(k * 16, 16)] * r2[j, pl.ds(k * 16, 16)]
          s = jnp.sum(acc)
          vec = jnp.where(lane == l, s, vec)
        out_v[pl.ds(base_j, 16)] = vec
        return c

      lax.fori_loop(0, B // 16, group_body, 0)
      pltpu.sync_copy(out_v, out_hbm.at[pl.ds(block_base(block), B)])

  def pair_body(t0, carry):
    step(t0 * 2, 0)
    step(t0 * 2 + 1, 1)
    return carry

  lax.fori_loop(0, (ITERS + 1) // 2, pair_body, 0)


@jax.jit
def _run(n1, n2, w):
  mesh = plsc.VectorSubcoreMesh(core_axis_name="c", subcore_axis_name="s")
  f = pl.kernel(
      _sc_body,
      mesh=mesh,
      compiler_params=pltpu.CompilerParams(needs_layout_passes=False),
      out_type=jax.ShapeDtypeStruct((E,), jnp.float32),
      scratch_types=[
          pltpu.VMEM((B,), jnp.int32),
          pltpu.VMEM((B,), jnp.int32),
          pltpu.VMEM((B,), jnp.int32),
          pltpu.VMEM((B,), jnp.int32),
          pltpu.VMEM((B, D), jnp.float32),
          pltpu.VMEM((B, D), jnp.float32),
          pltpu.VMEM((B, D), jnp.float32),
          pltpu.VMEM((B, D), jnp.float32),
          pltpu.VMEM((B,), jnp.float32),
          pltpu.SemaphoreType.DMA,
          pltpu.SemaphoreType.DMA,
          pltpu.SemaphoreType.DMA,
          pltpu.SemaphoreType.DMA,
      ],
  )
  return f(n1, n2, w)


def kernel(n1_indices, n2_indices, W):
  n1 = n1_indices.astype(jnp.int32)
  n2 = n2_indices.astype(jnp.int32)
  return _run(n1, n2, W)
